# baseline (speedup 1.0000x reference)
import functools

import jax
import jax.numpy as jnp
from jax import lax
from jax.experimental import pallas as pl
from jax.experimental.pallas import tpu as pltpu
from jax.experimental.pallas import tpu_sc as plsc

N = 10000
B = 50
DIN = 128
DH = 128
RESHAPE = 30
MAXN = 256

N_PAD = 10240
CHUNK = 128
NSUB = 16
E_PAD = 32 * 79 * CHUNK
ES = E_PAD // NSUB
NCHUNK = ES // CHUNK
ROWS16 = N_PAD // NSUB

J = 32
NG = 2 * 2 * B * J
GCH = 128
PERW = NG // 32
GCHUNKS = ((0, 128), (128, 72))

_mesh = plsc.VectorSubcoreMesh(core_axis_name="c", subcore_axis_name="s")


@functools.partial(
    pl.kernel, mesh=_mesh,
    out_type=jax.ShapeDtypeStruct((2, N_PAD, DH), jnp.float32),
    scratch_types=[
        pltpu.VMEM((NCHUNK, CHUNK), jnp.int32),
        pltpu.VMEM((CHUNK, DH), jnp.float32),
        pltpu.VMEM_SHARED((N_PAD, DH), jnp.float32),
    ],
)
def _sc_deg(dst_hbm, ones_hbm, zeros_hbm, out_hbm, didx_v, ones_v, acc_sh):
    c = lax.axis_index("c")
    s = lax.axis_index("s")
    pltpu.sync_copy(dst_hbm.at[c, s], didx_v)
    pltpu.sync_copy(zeros_hbm.at[pl.ds(s * ROWS16, ROWS16)],
                    acc_sh.at[pl.ds(s * ROWS16, ROWS16)])
    pltpu.sync_copy(ones_hbm, ones_v)
    plsc.subcore_barrier()

    def body(i, carry):
        pltpu.sync_copy(ones_v, acc_sh.at[didx_v.at[i]], add=True)
        return carry

    lax.fori_loop(0, NCHUNK, body, 0)
    plsc.subcore_barrier()
    pltpu.sync_copy(acc_sh.at[pl.ds(s * ROWS16, ROWS16)],
                    out_hbm.at[c, pl.ds(s * ROWS16, ROWS16)])


@functools.partial(
    pl.kernel, mesh=_mesh,
    out_type=jax.ShapeDtypeStruct((2, N_PAD, DH), jnp.float32),
    scratch_types=[
        pltpu.VMEM((2, CHUNK), jnp.int32),
        pltpu.VMEM((2, CHUNK), jnp.int32),
        pltpu.VMEM((CHUNK, DH), jnp.float32),
        pltpu.VMEM((CHUNK, DH), jnp.float32),
        pltpu.VMEM_SHARED((N_PAD, DH), jnp.float32),
        pltpu.SemaphoreType.DMA,
        pltpu.SemaphoreType.DMA,
        pltpu.SemaphoreType.DMA,
        pltpu.SemaphoreType.DMA,
    ],
)
def _sc_conv(hp_hbm, eidx_hbm, zeros_hbm, out_hbm,
             e0, e1, r0, r1, acc_sh, ise0, ise1, gse0, gse1):
    c = lax.axis_index("c")
    s = lax.axis_index("s")

    def idx_desc(i, ebuf, sem):
        return pltpu.make_async_copy(eidx_hbm.at[c, s, i], ebuf, sem)

    def g_desc(ebuf, rbuf, sem):
        return pltpu.make_async_copy(hp_hbm.at[ebuf.at[0]], rbuf, sem)

    idx_desc(0, e0, ise0).start()
    idx_desc(1, e1, ise1).start()
    pltpu.sync_copy(zeros_hbm.at[pl.ds(s * ROWS16, ROWS16)],
                    acc_sh.at[pl.ds(s * ROWS16, ROWS16)])
    plsc.subcore_barrier()
    idx_desc(0, e0, ise0).wait()
    g_desc(e0, r0, gse0).start()

    def body(i2, carry):
        i = 2 * i2
        g_desc(e0, r0, gse0).wait()
        idx_desc(i + 1, e1, ise1).wait()
        g_desc(e1, r1, gse1).start()
        pltpu.sync_copy(r0, acc_sh.at[e0.at[1]], add=True)

        @pl.when(i + 2 < NCHUNK)
        def _():
            idx_desc(i + 2, e0, ise0).start()

        g_desc(e1, r1, gse1).wait()

        @pl.when(i + 2 < NCHUNK)
        def _():
            idx_desc(i + 2, e0, ise0).wait()
            g_desc(e0, r0, gse0).start()

        pltpu.sync_copy(r1, acc_sh.at[e1.at[1]], add=True)

        @pl.when(i + 3 < NCHUNK)
        def _():
            idx_desc(i + 3, e1, ise1).start()

        return carry

    lax.fori_loop(0, NCHUNK // 2, body, 0)
    plsc.subcore_barrier()
    pltpu.sync_copy(acc_sh.at[pl.ds(s * ROWS16, ROWS16)],
                    out_hbm.at[c, pl.ds(s * ROWS16, ROWS16)])


@functools.partial(
    pl.kernel, mesh=_mesh,
    out_type=jax.ShapeDtypeStruct((7, NG, DH), jnp.float32),
    scratch_types=[
        pltpu.VMEM((2, CHUNK), jnp.int32),
        pltpu.VMEM((2, CHUNK), jnp.int32),
        pltpu.VMEM((CHUNK, DH), jnp.float32),
        pltpu.VMEM((CHUNK, DH), jnp.float32),
        pltpu.VMEM((PERW,), jnp.int32),
        pltpu.VMEM((PERW,), jnp.int32),
        pltpu.VMEM_SHARED((N_PAD, DH), jnp.float32),
        pltpu.SemaphoreType.DMA,
        pltpu.SemaphoreType.DMA,
        pltpu.SemaphoreType.DMA,
        pltpu.SemaphoreType.DMA,
    ],
)
def _sc_conv3(hp_hbm, eidx_hbm, zeros_hbm, agg1_hbm, hp1_hbm, agg2_hbm,
              hp2_hbm, deg_hbm, gidx_hbm, aggidx_hbm, rows_out,
              e0, e1, r0, r1, gi_v, ai_v, acc_sh, ise0, ise1, gse0, gse1):
    c = lax.axis_index("c")
    s = lax.axis_index("s")

    def idx_desc(i, ebuf, sem):
        return pltpu.make_async_copy(eidx_hbm.at[c, s, i], ebuf, sem)

    def g_desc(ebuf, rbuf, sem):
        return pltpu.make_async_copy(hp_hbm.at[ebuf.at[0]], rbuf, sem)

    idx_desc(0, e0, ise0).start()
    idx_desc(1, e1, ise1).start()
    pltpu.sync_copy(zeros_hbm.at[pl.ds(s * ROWS16, ROWS16)],
                    acc_sh.at[pl.ds(s * ROWS16, ROWS16)])
    plsc.subcore_barrier()
    idx_desc(0, e0, ise0).wait()
    g_desc(e0, r0, gse0).start()

    def body(i2, carry):
        i = 2 * i2
        g_desc(e0, r0, gse0).wait()
        idx_desc(i + 1, e1, ise1).wait()
        g_desc(e1, r1, gse1).start()
        pltpu.sync_copy(r0, acc_sh.at[e0.at[1]], add=True)

        @pl.when(i + 2 < NCHUNK)
        def _():
            idx_desc(i + 2, e0, ise0).start()

        g_desc(e1, r1, gse1).wait()

        @pl.when(i + 2 < NCHUNK)
        def _():
            idx_desc(i + 2, e0, ise0).wait()
            g_desc(e0, r0, gse0).start()

        pltpu.sync_copy(r1, acc_sh.at[e1.at[1]], add=True)

        @pl.when(i + 3 < NCHUNK)
        def _():
            idx_desc(i + 3, e1, ise1).start()

        return carry

    lax.fori_loop(0, NCHUNK // 2, body, 0)
    plsc.subcore_barrier()

    w2 = c * NSUB + s
    pltpu.sync_copy(gidx_hbm.at[w2], gi_v)
    pltpu.sync_copy(aggidx_hbm.at[w2], ai_v)
    jobs = ([(t, off, sz, t, False) for t in range(6)
             for (off, sz) in GCHUNKS]
            + [(6, off, sz, 0, True) for (off, sz) in GCHUNKS])
    tabs = (agg1_hbm, hp1_hbm, agg2_hbm, hp2_hbm, hp_hbm, deg_hbm)
    bufs = (r0, r1)
    sems = (gse0, gse1)

    def j_desc(k):
        t, off, sz, ti, from_acc = jobs[k]
        src = acc_sh if from_acc else tabs[ti]
        iv = ai_v if from_acc else gi_v
        return pltpu.make_async_copy(
            src.at[iv.at[pl.ds(off, sz)]],
            bufs[k % 2].at[pl.ds(0, sz)], sems[k % 2])

    j_desc(0).start()
    for k, (t, off, sz, ti, from_acc) in enumerate(jobs):
        j_desc(k).wait()
        if k + 1 < len(jobs):
            j_desc(k + 1).start()
        pltpu.sync_copy(bufs[k % 2].at[pl.ds(0, sz)],
                        rows_out.at[t, pl.ds(w2 * PERW + off, sz)])


_BLK = 256
_GRID = 2 * N_PAD // _BLK


def _mm1_body(x_ref, deg_ref, w_ref, hp_ref, dv_ref):
    dinv = lax.rsqrt(deg_ref[:, :1] + 1.0)
    h = jnp.dot(x_ref[...], w_ref[...], preferred_element_type=jnp.float32)
    hp_ref[...] = h * dinv
    dv_ref[...] = jnp.broadcast_to(dinv, (_BLK, 8))


def _mm1(x, deg, W):
    return pl.pallas_call(
        _mm1_body,
        grid=(_GRID,),
        in_specs=[
            pl.BlockSpec((_BLK, DIN), lambda i: (i, 0)),
            pl.BlockSpec((_BLK, DH), lambda i: (i, 0)),
            pl.BlockSpec((DIN, DH), lambda i: (0, 0)),
        ],
        out_specs=[
            pl.BlockSpec((_BLK, DH), lambda i: (i, 0)),
            pl.BlockSpec((_BLK, 8), lambda i: (i, 0)),
        ],
        out_shape=[
            jax.ShapeDtypeStruct((2 * N_PAD, DH), jnp.float32),
            jax.ShapeDtypeStruct((2 * N_PAD, 8), jnp.float32),
        ],
    )(x, deg, W)


def _epi_body(agg_ref, hp_ref, dv_ref, b_ref, w_ref, hpn_ref):
    dinv = dv_ref[:, :1]
    xl = jnp.maximum(dinv * (agg_ref[...] + hp_ref[...]) + b_ref[...], 0.0)
    hpn_ref[...] = jnp.dot(xl, w_ref[...], preferred_element_type=jnp.float32) * dinv


def _epi(agg, hp, dv, b, Wn):
    return pl.pallas_call(
        _epi_body,
        grid=(_GRID,),
        in_specs=[
            pl.BlockSpec((_BLK, DH), lambda i: (i, 0)),
            pl.BlockSpec((_BLK, DH), lambda i: (i, 0)),
            pl.BlockSpec((_BLK, 8), lambda i: (i, 0)),
            pl.BlockSpec((1, DH), lambda i: (0, 0)),
            pl.BlockSpec((DH, DH), lambda i: (0, 0)),
        ],
        out_specs=pl.BlockSpec((_BLK, DH), lambda i: (i, 0)),
        out_shape=jax.ShapeDtypeStruct((2 * N_PAD, DH), jnp.float32),
    )(agg, hp, dv, b, Wn)


def _rowepi3_body(agg_ref, hp_ref, deg_ref, b_ref, x_ref):
    dinv = lax.rsqrt(deg_ref[0][:, :1] + 1.0)
    x_ref[0] = jnp.maximum(dinv * (agg_ref[0] + hp_ref[0]) + b_ref[0], 0.0)


_REBLK = 256


def _rowepi3(rows7, b_stack):
    return pl.pallas_call(
        _rowepi3_body,
        grid=(3, NG // _REBLK),
        in_specs=[
            pl.BlockSpec((1, _REBLK, DH), lambda l, i: (2 * l + 2 * (l // 2), i, 0)),
            pl.BlockSpec((1, _REBLK, DH), lambda l, i: (2 * l + 1 - (l // 2), i, 0)),
            pl.BlockSpec((1, _REBLK, DH), lambda l, i: (5, i, 0)),
            pl.BlockSpec((1, 1, DH), lambda l, i: (l, 0, 0)),
        ],
        out_specs=pl.BlockSpec((1, _REBLK, DH), lambda l, i: (l, i, 0)),
        out_shape=jax.ShapeDtypeStruct((3, NG, DH), jnp.float32),
    )(rows7, rows7, rows7, b_stack)


def _bmm_body(a0, a1, b0, b1, w0s, w1s, w0t, w1t, o_ref):
    ys = w0s[0][:, :1] * a0[0] + w1s[0][:, :1] * a1[0]
    yt = w0t[0][:, :1] * b0[0] + w1t[0][:, :1] * b1[0]
    o_ref[0, 0] = lax.dot_general(ys, yt, (((1,), (1,)), ((), ())),
                                  preferred_element_type=jnp.float32)


def _bmm(a0, a1, b0, b1, w0s, w1s, w0t, w1t):
    row_spec = pl.BlockSpec((1, J, DH), lambda l, b: (l, b, 0))
    wt_spec = pl.BlockSpec((1, J, 8), lambda l, b: (b, 0, 0))
    return pl.pallas_call(
        _bmm_body,
        grid=(3, B),
        in_specs=[row_spec, row_spec, row_spec, row_spec,
                  wt_spec, wt_spec, wt_spec, wt_spec],
        out_specs=pl.BlockSpec((1, 1, J, J), lambda l, b: (b, l, 0, 0)),
        out_shape=jax.ShapeDtypeStruct((B, 3, J, J), jnp.float32),
    )(a0, a1, b0, b1, w0s, w1s, w0t, w1t)


def _pad_edges(ei, core):
    src, dst = ei[0].astype(jnp.int32), ei[1].astype(jnp.int32)
    npad = E_PAD - src.shape[0]
    padr = (N + (jnp.arange(npad, dtype=jnp.int32) % 16))
    src_p = jnp.concatenate([src, padr])
    dst_p = jnp.concatenate([dst, padr])
    return src_p + core * N_PAD, dst_p


def _interp_geometry():
    scale = MAXN / RESHAPE
    c = (jnp.arange(RESHAPE, dtype=jnp.float32) + 0.5) * scale - 0.5
    c0 = jnp.floor(c)
    w = c - c0
    i0 = jnp.clip(c0, 0, MAXN - 1).astype(jnp.int32)
    i1 = jnp.clip(c0 + 1, 0, MAXN - 1).astype(jnp.int32)
    return i0, i1, w


def _proj_idx(batch, max_num_nodes, base, i0, i1, w):
    bd = jnp.searchsorted(batch, jnp.arange(B + 1, dtype=batch.dtype)).astype(jnp.int32)
    starts, counts = bd[:B], bd[1:] - bd[:B]
    lim = jnp.minimum(counts, max_num_nodes)
    v0 = i0[None, :] < lim[:, None]
    v1 = i1[None, :] < lim[:, None]
    spread = (jnp.arange(B, dtype=jnp.int32)[:, None] * J
              + jnp.arange(RESHAPE, dtype=jnp.int32)[None, :])
    g0 = jnp.where(v0, starts[:, None] + i0[None, :], spread) + base
    g1 = jnp.where(v1, starts[:, None] + i1[None, :], spread) + base
    w0 = jnp.where(v0, 1.0 - w[None, :], 0.0)
    w1 = jnp.where(v1, w[None, :], 0.0)
    pad2 = ((0, 0), (0, J - RESHAPE))
    g0 = jnp.pad(g0, pad2, constant_values=base)
    g1 = jnp.pad(g1, pad2, constant_values=base)
    w0 = jnp.pad(w0, pad2)
    w1 = jnp.pad(w1, pad2)
    return (g0.reshape(-1), g1.reshape(-1),
            jnp.broadcast_to(w0.reshape(B, J, 1), (B, J, 8)),
            jnp.broadcast_to(w1.reshape(B, J, 1), (B, J, 8)))


def kernel(x_s, edge_index_s, batch_s, x_t, edge_index_t, batch_t,
           W1, b1, W2, b2, W3, b3, max_num_nodes):
    f32 = jnp.float32
    x_both = jnp.zeros((2 * N_PAD, DIN), f32)
    x_both = x_both.at[:N].set(x_s).at[N_PAD:N_PAD + N].set(x_t)

    src_s, dst_s = _pad_edges(edge_index_s, 0)
    src_t, dst_t = _pad_edges(edge_index_t, 1)
    src_a = jnp.stack([src_s, src_t]).reshape(2, NSUB, NCHUNK, 1, CHUNK)
    dst_a = jnp.stack([dst_s, dst_t]).reshape(2, NSUB, NCHUNK, 1, CHUNK)
    eidx = jnp.concatenate([src_a, dst_a], axis=3)
    dst_g = dst_a.reshape(2, NSUB, NCHUNK, CHUNK)

    ones128 = jnp.ones((CHUNK, DH), f32)
    zeros128 = jnp.zeros((N_PAD, DH), f32)

    deg = _sc_deg(dst_g, ones128, zeros128).reshape(2 * N_PAD, DH)

    b1r = b1.reshape(1, DH)
    b2r = b2.reshape(1, DH)
    b3r = b3.reshape(1, DH)

    hp1, dv = _mm1(x_both, deg, W1)
    agg1 = _sc_conv(hp1, eidx, zeros128).reshape(2 * N_PAD, DH)
    hp2 = _epi(agg1, hp1, dv, b1r, W2)
    agg2 = _sc_conv(hp2, eidx, zeros128).reshape(2 * N_PAD, DH)
    hp3 = _epi(agg2, hp2, dv, b2r, W3)

    i0, i1, w = _interp_geometry()
    g0s, g1s, w0s, w1s = _proj_idx(batch_s, max_num_nodes, 0, i0, i1, w)
    g0t, g1t, w0t, w1t = _proj_idx(batch_t, max_num_nodes, N_PAD, i0, i1, w)
    gidx = jnp.concatenate([g0s, g1s, g0t, g1t]).reshape(32, PERW)
    aggidx = gidx - jnp.where(jnp.arange(32) < NSUB, 0, N_PAD)[:, None].astype(jnp.int32)

    rows7 = _sc_conv3(hp3, eidx, zeros128, agg1, hp1, agg2, hp2, deg, gidx, aggidx)
    b_stack = jnp.stack([b1r, b2r, b3r])
    rows = _rowepi3(rows7, b_stack)
    nbj = B * J
    a0 = rows[:, 0 * nbj:1 * nbj]
    a1 = rows[:, 1 * nbj:2 * nbj]
    c0 = rows[:, 2 * nbj:3 * nbj]
    c1 = rows[:, 3 * nbj:4 * nbj]

    out_full = _bmm(a0, a1, c0, c1, w0s, w1s, w0t, w1t)
    return out_full[:, :, :RESHAPE, :RESHAPE]


# speedup vs baseline: 1.1365x; 1.1365x over previous
import functools

import jax
import jax.numpy as jnp
from jax import lax
from jax.experimental import pallas as pl
from jax.experimental.pallas import tpu as pltpu
from jax.experimental.pallas import tpu_sc as plsc

N = 10000
B = 50
DIN = 128
DH = 128
RESHAPE = 30
MAXN = 256

N_PAD = 10240
CHUNK = 128
NSUB = 16
E_PAD = 32 * 79 * CHUNK
ES = E_PAD // NSUB
NCHUNK = ES // CHUNK
ROWS16 = N_PAD // NSUB

J = 32
NG = 2 * 2 * B * J
GCH = 128
PERW = NG // 32
GCHUNKS = ((0, 128), (128, 72))

_mesh = plsc.VectorSubcoreMesh(core_axis_name="c", subcore_axis_name="s")


@functools.partial(
    pl.kernel, mesh=_mesh,
    out_type=jax.ShapeDtypeStruct((2, N_PAD, DH), jnp.float32),
    scratch_types=[
        pltpu.VMEM((NCHUNK, CHUNK), jnp.int32),
        pltpu.VMEM((CHUNK, DH), jnp.float32),
        pltpu.VMEM_SHARED((N_PAD, DH), jnp.float32),
    ],
)
def _sc_deg(dst_hbm, ones_hbm, zeros_hbm, out_hbm, didx_v, ones_v, acc_sh):
    c = lax.axis_index("c")
    s = lax.axis_index("s")
    pltpu.sync_copy(dst_hbm.at[c, s], didx_v)
    pltpu.sync_copy(zeros_hbm.at[pl.ds(s * ROWS16, ROWS16)],
                    acc_sh.at[pl.ds(s * ROWS16, ROWS16)])
    pltpu.sync_copy(ones_hbm, ones_v)
    plsc.subcore_barrier()

    def body(i, carry):
        pltpu.sync_copy(ones_v, acc_sh.at[didx_v.at[i]], add=True)
        return carry

    lax.fori_loop(0, NCHUNK, body, 0)
    plsc.subcore_barrier()
    pltpu.sync_copy(acc_sh.at[pl.ds(s * ROWS16, ROWS16)],
                    out_hbm.at[c, pl.ds(s * ROWS16, ROWS16)])


@functools.partial(
    pl.kernel, mesh=_mesh,
    out_type=jax.ShapeDtypeStruct((2, N_PAD, DH), jnp.float32),
    scratch_types=[
        pltpu.VMEM((2, CHUNK), jnp.int32),
        pltpu.VMEM((2, CHUNK), jnp.int32),
        pltpu.VMEM((CHUNK, DH), jnp.float32),
        pltpu.VMEM((CHUNK, DH), jnp.float32),
        pltpu.VMEM_SHARED((N_PAD, DH), jnp.float32),
        pltpu.SemaphoreType.DMA,
        pltpu.SemaphoreType.DMA,
        pltpu.SemaphoreType.DMA,
        pltpu.SemaphoreType.DMA,
    ],
)
def _sc_conv(hp_hbm, eidx_hbm, zeros_hbm, out_hbm,
             e0, e1, r0, r1, acc_sh, ise0, ise1, gse0, gse1):
    c = lax.axis_index("c")
    s = lax.axis_index("s")

    def idx_desc(i, ebuf, sem):
        return pltpu.make_async_copy(eidx_hbm.at[c, s, i], ebuf, sem)

    def g_desc(ebuf, rbuf, sem):
        return pltpu.make_async_copy(hp_hbm.at[ebuf.at[0]], rbuf, sem)

    idx_desc(0, e0, ise0).start()
    idx_desc(1, e1, ise1).start()
    pltpu.sync_copy(zeros_hbm.at[pl.ds(s * ROWS16, ROWS16)],
                    acc_sh.at[pl.ds(s * ROWS16, ROWS16)])
    plsc.subcore_barrier()
    idx_desc(0, e0, ise0).wait()
    g_desc(e0, r0, gse0).start()

    def body(i2, carry):
        i = 2 * i2
        g_desc(e0, r0, gse0).wait()
        idx_desc(i + 1, e1, ise1).wait()
        g_desc(e1, r1, gse1).start()
        pltpu.sync_copy(r0, acc_sh.at[e0.at[1]], add=True)

        @pl.when(i + 2 < NCHUNK)
        def _():
            idx_desc(i + 2, e0, ise0).start()

        g_desc(e1, r1, gse1).wait()

        @pl.when(i + 2 < NCHUNK)
        def _():
            idx_desc(i + 2, e0, ise0).wait()
            g_desc(e0, r0, gse0).start()

        pltpu.sync_copy(r1, acc_sh.at[e1.at[1]], add=True)

        @pl.when(i + 3 < NCHUNK)
        def _():
            idx_desc(i + 3, e1, ise1).start()

        return carry

    lax.fori_loop(0, NCHUNK // 2, body, 0)
    plsc.subcore_barrier()
    pltpu.sync_copy(acc_sh.at[pl.ds(s * ROWS16, ROWS16)],
                    out_hbm.at[c, pl.ds(s * ROWS16, ROWS16)])


@functools.partial(
    pl.kernel, mesh=_mesh,
    out_type=jax.ShapeDtypeStruct((7, NG, DH), jnp.float32),
    scratch_types=[
        pltpu.VMEM((2, CHUNK), jnp.int32),
        pltpu.VMEM((2, CHUNK), jnp.int32),
        pltpu.VMEM((CHUNK, DH), jnp.float32),
        pltpu.VMEM((CHUNK, DH), jnp.float32),
        pltpu.VMEM((PERW,), jnp.int32),
        pltpu.VMEM((PERW,), jnp.int32),
        pltpu.VMEM_SHARED((N_PAD, DH), jnp.float32),
        pltpu.SemaphoreType.DMA,
        pltpu.SemaphoreType.DMA,
        pltpu.SemaphoreType.DMA,
        pltpu.SemaphoreType.DMA,
    ],
)
def _sc_conv3(hp_hbm, eidx_hbm, zeros_hbm, agg1_hbm, hp1_hbm, agg2_hbm,
              hp2_hbm, deg_hbm, gidx_hbm, aggidx_hbm, rows_out,
              e0, e1, r0, r1, gi_v, ai_v, acc_sh, ise0, ise1, gse0, gse1):
    c = lax.axis_index("c")
    s = lax.axis_index("s")

    def idx_desc(i, ebuf, sem):
        return pltpu.make_async_copy(eidx_hbm.at[c, s, i], ebuf, sem)

    def g_desc(ebuf, rbuf, sem):
        return pltpu.make_async_copy(hp_hbm.at[ebuf.at[0]], rbuf, sem)

    idx_desc(0, e0, ise0).start()
    idx_desc(1, e1, ise1).start()
    pltpu.sync_copy(zeros_hbm.at[pl.ds(s * ROWS16, ROWS16)],
                    acc_sh.at[pl.ds(s * ROWS16, ROWS16)])
    plsc.subcore_barrier()
    idx_desc(0, e0, ise0).wait()
    g_desc(e0, r0, gse0).start()

    def body(i2, carry):
        i = 2 * i2
        g_desc(e0, r0, gse0).wait()
        idx_desc(i + 1, e1, ise1).wait()
        g_desc(e1, r1, gse1).start()
        pltpu.sync_copy(r0, acc_sh.at[e0.at[1]], add=True)

        @pl.when(i + 2 < NCHUNK)
        def _():
            idx_desc(i + 2, e0, ise0).start()

        g_desc(e1, r1, gse1).wait()

        @pl.when(i + 2 < NCHUNK)
        def _():
            idx_desc(i + 2, e0, ise0).wait()
            g_desc(e0, r0, gse0).start()

        pltpu.sync_copy(r1, acc_sh.at[e1.at[1]], add=True)

        @pl.when(i + 3 < NCHUNK)
        def _():
            idx_desc(i + 3, e1, ise1).start()

        return carry

    lax.fori_loop(0, NCHUNK // 2, body, 0)
    plsc.subcore_barrier()

    w2 = c * NSUB + s
    pltpu.sync_copy(gidx_hbm.at[w2], gi_v)
    pltpu.sync_copy(aggidx_hbm.at[w2], ai_v)
    jobs = ([(t, off, sz, t, False) for t in range(6)
             for (off, sz) in GCHUNKS]
            + [(6, off, sz, 0, True) for (off, sz) in GCHUNKS])
    tabs = (agg1_hbm, hp1_hbm, agg2_hbm, hp2_hbm, hp_hbm, deg_hbm)
    bufs = (r0, r1)
    sems = (gse0, gse1)

    def j_desc(k):
        t, off, sz, ti, from_acc = jobs[k]
        src = acc_sh if from_acc else tabs[ti]
        iv = ai_v if from_acc else gi_v
        return pltpu.make_async_copy(
            src.at[iv.at[pl.ds(off, sz)]],
            bufs[k % 2].at[pl.ds(0, sz)], sems[k % 2])

    j_desc(0).start()
    for k, (t, off, sz, ti, from_acc) in enumerate(jobs):
        j_desc(k).wait()
        if k + 1 < len(jobs):
            j_desc(k + 1).start()
        pltpu.sync_copy(bufs[k % 2].at[pl.ds(0, sz)],
                        rows_out.at[t, pl.ds(w2 * PERW + off, sz)])


_BLK = 512
_GRID = 2 * N_PAD // _BLK


def _mm1_body(x_ref, deg_ref, w_ref, hp_ref, dv_ref):
    dinv = lax.rsqrt(deg_ref[:, :1] + 1.0)
    h = jnp.dot(x_ref[...], w_ref[...], preferred_element_type=jnp.float32)
    hp_ref[...] = h * dinv
    dv_ref[...] = jnp.broadcast_to(dinv, (_BLK, 8))


def _mm1(x, deg, W):
    return pl.pallas_call(
        _mm1_body,
        grid=(_GRID,),
        in_specs=[
            pl.BlockSpec((_BLK, DIN), lambda i: (i, 0)),
            pl.BlockSpec((_BLK, DH), lambda i: (i, 0)),
            pl.BlockSpec((DIN, DH), lambda i: (0, 0)),
        ],
        out_specs=[
            pl.BlockSpec((_BLK, DH), lambda i: (i, 0)),
            pl.BlockSpec((_BLK, 8), lambda i: (i, 0)),
        ],
        out_shape=[
            jax.ShapeDtypeStruct((2 * N_PAD, DH), jnp.float32),
            jax.ShapeDtypeStruct((2 * N_PAD, 8), jnp.float32),
        ],
    )(x, deg, W)


def _epi_body(agg_ref, hp_ref, dv_ref, b_ref, w_ref, hpn_ref):
    dinv = dv_ref[:, :1]
    xl = jnp.maximum(dinv * (agg_ref[...] + hp_ref[...]) + b_ref[...], 0.0)
    hpn_ref[...] = jnp.dot(xl, w_ref[...], preferred_element_type=jnp.float32) * dinv


def _epi(agg, hp, dv, b, Wn):
    return pl.pallas_call(
        _epi_body,
        grid=(_GRID,),
        in_specs=[
            pl.BlockSpec((_BLK, DH), lambda i: (i, 0)),
            pl.BlockSpec((_BLK, DH), lambda i: (i, 0)),
            pl.BlockSpec((_BLK, 8), lambda i: (i, 0)),
            pl.BlockSpec((1, DH), lambda i: (0, 0)),
            pl.BlockSpec((DH, DH), lambda i: (0, 0)),
        ],
        out_specs=pl.BlockSpec((_BLK, DH), lambda i: (i, 0)),
        out_shape=jax.ShapeDtypeStruct((2 * N_PAD, DH), jnp.float32),
    )(agg, hp, dv, b, Wn)


def _rowepi3_body(rows_ref, b_ref, x_ref):
    dinv = lax.rsqrt(rows_ref[5][:, :1] + 1.0)
    for l, (ta, th) in enumerate(((0, 1), (2, 3), (6, 4))):
        x_ref[l] = jnp.maximum(
            dinv * (rows_ref[ta] + rows_ref[th]) + b_ref[l], 0.0)


_REBLK = 256


def _rowepi3(rows7, b_stack):
    return pl.pallas_call(
        _rowepi3_body,
        grid=(NG // _REBLK,),
        in_specs=[
            pl.BlockSpec((7, _REBLK, DH), lambda i: (0, i, 0)),
            pl.BlockSpec((3, 1, DH), lambda i: (0, 0, 0)),
        ],
        out_specs=pl.BlockSpec((3, _REBLK, DH), lambda i: (0, i, 0)),
        out_shape=jax.ShapeDtypeStruct((3, NG, DH), jnp.float32),
    )(rows7, b_stack)


def _bmm_body(a0, a1, c0, c1, w0s, w1s, w0t, w1t, o_ref):
    for l in range(3):
        ys = w0s[0][:, :1] * a0[l] + w1s[0][:, :1] * a1[l]
        yt = w0t[0][:, :1] * c0[l] + w1t[0][:, :1] * c1[l]
        o_ref[0, l] = lax.dot_general(ys, yt, (((1,), (1,)), ((), ())),
                                      preferred_element_type=jnp.float32)


def _bmm(rows, w0s, w1s, w0t, w1t):
    def row_spec(off):
        return pl.BlockSpec((3, J, DH), lambda b: (0, b + off, 0))

    wt_spec = pl.BlockSpec((1, J, 8), lambda b: (b, 0, 0))
    return pl.pallas_call(
        _bmm_body,
        grid=(B,),
        in_specs=[row_spec(0), row_spec(B), row_spec(2 * B), row_spec(3 * B),
                  wt_spec, wt_spec, wt_spec, wt_spec],
        out_specs=pl.BlockSpec((1, 3, J, J), lambda b: (b, 0, 0, 0)),
        out_shape=jax.ShapeDtypeStruct((B, 3, J, J), jnp.float32),
    )(rows, rows, rows, rows, w0s, w1s, w0t, w1t)


def _pad_edges(ei, core):
    src, dst = ei[0].astype(jnp.int32), ei[1].astype(jnp.int32)
    npad = E_PAD - src.shape[0]
    padr = (N + (jnp.arange(npad, dtype=jnp.int32) % 16))
    src_p = jnp.concatenate([src, padr])
    dst_p = jnp.concatenate([dst, padr])
    return src_p + core * N_PAD, dst_p


def _interp_geometry():
    scale = MAXN / RESHAPE
    c = (jnp.arange(RESHAPE, dtype=jnp.float32) + 0.5) * scale - 0.5
    c0 = jnp.floor(c)
    w = c - c0
    i0 = jnp.clip(c0, 0, MAXN - 1).astype(jnp.int32)
    i1 = jnp.clip(c0 + 1, 0, MAXN - 1).astype(jnp.int32)
    return i0, i1, w


def _proj_idx(batch, max_num_nodes, base, i0, i1, w):
    bd = jnp.searchsorted(batch, jnp.arange(B + 1, dtype=batch.dtype)).astype(jnp.int32)
    starts, counts = bd[:B], bd[1:] - bd[:B]
    lim = jnp.minimum(counts, max_num_nodes)
    v0 = i0[None, :] < lim[:, None]
    v1 = i1[None, :] < lim[:, None]
    spread = (jnp.arange(B, dtype=jnp.int32)[:, None] * J
              + jnp.arange(RESHAPE, dtype=jnp.int32)[None, :])
    g0 = jnp.where(v0, starts[:, None] + i0[None, :], spread) + base
    g1 = jnp.where(v1, starts[:, None] + i1[None, :], spread) + base
    w0 = jnp.where(v0, 1.0 - w[None, :], 0.0)
    w1 = jnp.where(v1, w[None, :], 0.0)
    pad2 = ((0, 0), (0, J - RESHAPE))
    g0 = jnp.pad(g0, pad2, constant_values=base)
    g1 = jnp.pad(g1, pad2, constant_values=base)
    w0 = jnp.pad(w0, pad2)
    w1 = jnp.pad(w1, pad2)
    return (g0.reshape(-1), g1.reshape(-1),
            jnp.broadcast_to(w0.reshape(B, J, 1), (B, J, 8)),
            jnp.broadcast_to(w1.reshape(B, J, 1), (B, J, 8)))


def kernel(x_s, edge_index_s, batch_s, x_t, edge_index_t, batch_t,
           W1, b1, W2, b2, W3, b3, max_num_nodes):
    f32 = jnp.float32
    x_both = jnp.zeros((2 * N_PAD, DIN), f32)
    x_both = x_both.at[:N].set(x_s).at[N_PAD:N_PAD + N].set(x_t)

    src_s, dst_s = _pad_edges(edge_index_s, 0)
    src_t, dst_t = _pad_edges(edge_index_t, 1)
    src_a = jnp.stack([src_s, src_t]).reshape(2, NSUB, NCHUNK, 1, CHUNK)
    dst_a = jnp.stack([dst_s, dst_t]).reshape(2, NSUB, NCHUNK, 1, CHUNK)
    eidx = jnp.concatenate([src_a, dst_a], axis=3)
    dst_g = dst_a.reshape(2, NSUB, NCHUNK, CHUNK)

    ones128 = jnp.ones((CHUNK, DH), f32)
    zeros128 = jnp.zeros((N_PAD, DH), f32)

    deg = _sc_deg(dst_g, ones128, zeros128).reshape(2 * N_PAD, DH)

    b1r = b1.reshape(1, DH)
    b2r = b2.reshape(1, DH)
    b3r = b3.reshape(1, DH)

    hp1, dv = _mm1(x_both, deg, W1)
    agg1 = _sc_conv(hp1, eidx, zeros128).reshape(2 * N_PAD, DH)
    hp2 = _epi(agg1, hp1, dv, b1r, W2)
    agg2 = _sc_conv(hp2, eidx, zeros128).reshape(2 * N_PAD, DH)
    hp3 = _epi(agg2, hp2, dv, b2r, W3)

    i0, i1, w = _interp_geometry()
    g0s, g1s, w0s, w1s = _proj_idx(batch_s, max_num_nodes, 0, i0, i1, w)
    g0t, g1t, w0t, w1t = _proj_idx(batch_t, max_num_nodes, N_PAD, i0, i1, w)
    gidx = jnp.concatenate([g0s, g1s, g0t, g1t]).reshape(32, PERW)
    aggidx = gidx - jnp.where(jnp.arange(32) < NSUB, 0, N_PAD)[:, None].astype(jnp.int32)

    rows7 = _sc_conv3(hp3, eidx, zeros128, agg1, hp1, agg2, hp2, deg, gidx, aggidx)
    b_stack = jnp.stack([b1r, b2r, b3r])
    rows = _rowepi3(rows7, b_stack)

    out_full = _bmm(rows, w0s, w1s, w0t, w1t)
    return out_full[:, :, :RESHAPE, :RESHAPE]
